# baseline (speedup 1.0000x reference)
import functools
import math

import jax
import jax.numpy as jnp
from jax import lax
from jax.experimental import pallas as pl
from jax.experimental.pallas import tpu as pltpu
from jax.experimental.pallas import tpu_sc as plsc

D_MODEL = 64
CHUNK = 128
LANES = 16
SCALE = math.sqrt(D_MODEL)
NBUF = 4
PF = 2

_MESH = dict(core_axis_name="c", subcore_axis_name="s")


def _splat(val):
    return jnp.full((LANES,), 0, jnp.int32) + val


@jax.jit
def _run(x_t, table_t, tail_pairs):
    info = plsc.get_sparse_core_info()
    nc, ns = info.num_cores, info.num_subcores
    nw = nc * ns
    seq, batch = x_t.shape
    vocab = table_t.shape[1]
    full_slabs = vocab // CHUNK
    spw = full_slabs // nw
    tail_full = full_slabs - spw * nw
    half = vocab - full_slabs * CHUNK

    @functools.partial(
        pl.kernel,
        mesh=plsc.VectorSubcoreMesh(**_MESH),
        compiler_params=pltpu.CompilerParams(
            use_tc_tiling_on_sc=True, needs_layout_passes=False
        ),
        out_type=jax.ShapeDtypeStruct((vocab // 2, CHUNK), jnp.float32),
        scratch_types=(
            [pltpu.VMEM((D_MODEL, CHUNK), jnp.float32) for _ in range(NBUF)]
            + [pltpu.VMEM((D_MODEL, CHUNK), jnp.float32) for _ in range(NBUF)]
            + [pltpu.SemaphoreType.DMA for _ in range(2 * NBUF)]
        ),
    )
    def fmt(tt_hbm, tailp_hbm, pr_hbm, *bufs_and_sems):
        ibuf = bufs_and_sems[:NBUF]
        obuf = bufs_and_sems[NBUF : 2 * NBUF]
        lsem = bufs_and_sems[2 * NBUF : 3 * NBUF]
        ssem = bufs_and_sems[3 * NBUF : 4 * NBUF]
        wid = lax.axis_index("s") * nc + lax.axis_index("c")

        def slab_of(t):
            return t * nw + wid

        def start_load(b, t):
            pltpu.make_async_copy(
                tt_hbm.at[:, pl.ds(slab_of(t) * CHUNK, CHUNK)], ibuf[b], lsem[b]
            ).start()

        def wait_load(b):
            pltpu.make_async_copy(
                tt_hbm.at[:, pl.ds(0, CHUNK)], ibuf[b], lsem[b]
            ).wait()

        def start_store(b, t):
            pltpu.make_async_copy(
                pr_hbm.at[pl.ds(slab_of(t) * D_MODEL, D_MODEL)], obuf[b], ssem[b]
            ).start()

        def wait_store(b):
            pltpu.make_async_copy(
                pr_hbm.at[pl.ds(0, D_MODEL)], obuf[b], ssem[b]
            ).wait()

        def transpose_slab(src, dst, nrows):
            def body(p, carry):
                for g in range(CHUNK // LANES):
                    row_ids = jax.lax.iota(jnp.int32, LANES) + (g % 4) * LANES
                    col_ids = _splat(2 * p + g // 4)
                    vals = plsc.load_gather(src, [row_ids, col_ids])
                    dst[p, pl.ds(g * LANES, LANES)] = vals
                return carry

            lax.fori_loop(0, nrows, body, 0)

        def outer(t, carry):
            pltpu.sync_copy(
                tt_hbm.at[:, pl.ds(slab_of(t) * CHUNK, CHUNK)], ibuf[0]
            )
            transpose_slab(ibuf[0], obuf[0], D_MODEL)
            pltpu.sync_copy(
                obuf[0], pr_hbm.at[pl.ds(slab_of(t) * D_MODEL, D_MODEL)]
            )
            return carry

        lax.fori_loop(0, spw, outer, 0)

        @pl.when(wid < tail_full)
        def _():
            sl = full_slabs - tail_full + wid
            pltpu.sync_copy(tt_hbm.at[:, pl.ds(sl * CHUNK, CHUNK)], ibuf[0])
            transpose_slab(ibuf[0], obuf[0], D_MODEL)
            pltpu.sync_copy(obuf[0], pr_hbm.at[pl.ds(sl * D_MODEL, D_MODEL)])

        if half:
            @pl.when(wid == nw - 1)
            def _():
                pltpu.sync_copy(tailp_hbm, obuf[1].at[pl.ds(0, half // 2)])
                pltpu.sync_copy(
                    obuf[1].at[pl.ds(0, half // 2)],
                    pr_hbm.at[pl.ds(full_slabs * D_MODEL, half // 2)],
                )

    @functools.partial(
        pl.kernel,
        mesh=plsc.VectorSubcoreMesh(**_MESH),
        compiler_params=pltpu.CompilerParams(
            use_tc_tiling_on_sc=True, needs_layout_passes=False
        ),
        out_type=jax.ShapeDtypeStruct((seq, D_MODEL, batch), jnp.float32),
        scratch_types=(
            [pltpu.VMEM((seq, CHUNK), jnp.int32)]
            + [pltpu.VMEM((NBUF, CHUNK), jnp.int32)]
            + [pltpu.VMEM((CHUNK, CHUNK), jnp.float32) for _ in range(NBUF)]
            + [pltpu.VMEM((D_MODEL, CHUNK), jnp.float32) for _ in range(NBUF)]
            + [pltpu.SemaphoreType.DMA for _ in range(2 * NBUF)]
        ),
    )
    def emb(x_hbm, tbl_hbm, out_hbm, idx_v, pidx_v, *bufs_and_sems):
        gbuf = bufs_and_sems[:NBUF]
        obuf = bufs_and_sems[NBUF : 2 * NBUF]
        gsem = bufs_and_sems[2 * NBUF : 3 * NBUF]
        ssem = bufs_and_sems[3 * NBUF : 4 * NBUF]
        wid = lax.axis_index("s") * nc + lax.axis_index("c")
        col0 = wid * CHUNK
        pltpu.sync_copy(x_hbm.at[:, pl.ds(col0, CHUNK)], idx_v)

        def start_gather(b, j):
            for g in range(CHUNK // LANES):
                s = pl.ds(g * LANES, LANES)
                pidx_v[b, s] = lax.shift_right_logical(idx_v[j, s], 1)
            pltpu.make_async_copy(
                tbl_hbm.at[pidx_v.at[b]], gbuf[b], gsem[b]
            ).start()

        def wait_gather(b):
            pltpu.make_async_copy(
                tbl_hbm.at[pidx_v.at[b]], gbuf[b], gsem[b]
            ).wait()

        def start_store(b, j):
            pltpu.make_async_copy(
                obuf[b], out_hbm.at[j, :, pl.ds(col0, CHUNK)], ssem[b]
            ).start()

        def wait_store(b):
            pltpu.make_async_copy(
                obuf[b], out_hbm.at[0, :, pl.ds(col0, CHUNK)], ssem[b]
            ).wait()

        for b in range(PF):
            start_gather(b, b)

        def outer(gi, carry):
            for b in range(NBUF):
                j = gi * NBUF + b
                jp = j + PF
                bp = (b + PF) % NBUF

                @pl.when(jp < seq)
                def _():
                    start_gather(bp, jp)

                wait_gather(b)

                @pl.when(j >= NBUF)
                def _():
                    wait_store(b)

                for g in range(CHUNK // LANES):
                    s = pl.ds(g * LANES, LANES)
                    row_ids = jax.lax.iota(jnp.int32, LANES) + (g * LANES)
                    col_base = (idx_v[j, s] & 1) * D_MODEL

                    @plsc.parallel_loop(0, D_MODEL, 1, unroll=4)
                    def _(k):
                        vals = plsc.load_gather(
                            gbuf[b], [row_ids, col_base + k]
                        )
                        obuf[b][k, s] = vals * SCALE

                start_store(b, j)
            return carry

        lax.fori_loop(0, seq // NBUF, outer, 0)
        for b in range(NBUF):
            wait_store(b)

    pair_rows = fmt(table_t, tail_pairs)
    return emb(x_t, pair_rows)


def kernel(x, table):
    b, s = x.shape
    vocab = table.shape[0]
    x_t = jnp.swapaxes(x, 0, 1).astype(jnp.int32)
    table_t = jnp.swapaxes(table, 0, 1)
    half = vocab - (vocab // CHUNK) * CHUNK
    tail_pairs = table[vocab - half :].reshape(half // 2, 2 * D_MODEL)
    out_t = _run(x_t, table_t, tail_pairs)
    return jnp.transpose(out_t, (2, 0, 1))


# speedup vs baseline: 1.6947x; 1.6947x over previous
import functools
import math

import jax
import jax.numpy as jnp
from jax import lax
from jax.experimental import pallas as pl
from jax.experimental.pallas import tpu as pltpu
from jax.experimental.pallas import tpu_sc as plsc

D_MODEL = 64
CHUNK = 128
LANES = 16
SCALE = math.sqrt(D_MODEL)
NBUF = 4
PF = 2

_MESH = dict(core_axis_name="c", subcore_axis_name="s")


def _splat(val):
    return jnp.full((LANES,), 0, jnp.int32) + val


@jax.jit
def _run(x_t, table_t, tail_pairs):
    info = plsc.get_sparse_core_info()
    nc, ns = info.num_cores, info.num_subcores
    nw = nc * ns
    seq, batch = x_t.shape
    vocab = table_t.shape[1]
    full_slabs = vocab // CHUNK
    spw = full_slabs // nw
    tail_full = full_slabs - spw * nw
    half = vocab - full_slabs * CHUNK

    @functools.partial(
        pl.kernel,
        mesh=plsc.VectorSubcoreMesh(**_MESH),
        compiler_params=pltpu.CompilerParams(
            use_tc_tiling_on_sc=True, needs_layout_passes=False
        ),
        out_type=jax.ShapeDtypeStruct((vocab // 2, CHUNK), jnp.float32),
        scratch_types=(
            [pltpu.VMEM((D_MODEL, CHUNK), jnp.float32) for _ in range(NBUF)]
            + [pltpu.VMEM((D_MODEL, CHUNK), jnp.float32) for _ in range(NBUF)]
            + [pltpu.SemaphoreType.DMA for _ in range(2 * NBUF)]
        ),
    )
    def fmt(tt_hbm, tailp_hbm, pr_hbm, *bufs_and_sems):
        ibuf = bufs_and_sems[:NBUF]
        obuf = bufs_and_sems[NBUF : 2 * NBUF]
        lsem = bufs_and_sems[2 * NBUF : 3 * NBUF]
        ssem = bufs_and_sems[3 * NBUF : 4 * NBUF]
        wid = lax.axis_index("s") * nc + lax.axis_index("c")

        def slab_of(t):
            return t * nw + wid

        def load_copy(b, t):
            return pltpu.make_async_copy(
                tt_hbm.at[:, pl.ds(slab_of(t) * CHUNK, CHUNK)], ibuf[b], lsem[b]
            )

        def store_copy(b, t):
            return pltpu.make_async_copy(
                obuf[b], pr_hbm.at[pl.ds(slab_of(t) * D_MODEL, D_MODEL)], ssem[b]
            )

        def transpose_slab(src, dst, nrows):
            @plsc.parallel_loop(0, nrows, 1, unroll=4)
            def _(p):
                for g in range(CHUNK // LANES):
                    row_ids = jax.lax.iota(jnp.int32, LANES) + (g % 4) * LANES
                    col_ids = _splat(2 * p + g // 4)
                    vals = plsc.load_gather(src, [row_ids, col_ids])
                    dst[p, pl.ds(g * LANES, LANES)] = vals

        for b in range(PF):
            load_copy(b, b).start()

        def outer(gi, carry):
            for b in range(NBUF):
                t = gi * NBUF + b
                tp = t + PF
                bp = (b + PF) % NBUF

                @pl.when(tp < spw)
                def _():
                    load_copy(bp, tp).start()

                load_copy(b, t).wait()

                @pl.when(t >= NBUF)
                def _():
                    store_copy(b, t - NBUF).wait()

                transpose_slab(ibuf[b], obuf[b], D_MODEL)
                store_copy(b, t).start()
            return carry

        lax.fori_loop(0, spw // NBUF, outer, 0)
        for b in range(NBUF):
            store_copy(b, spw - NBUF + b).wait()

        @pl.when(wid < tail_full)
        def _():
            sl = full_slabs - tail_full + wid
            pltpu.sync_copy(tt_hbm.at[:, pl.ds(sl * CHUNK, CHUNK)], ibuf[0])
            transpose_slab(ibuf[0], obuf[0], D_MODEL)
            pltpu.sync_copy(obuf[0], pr_hbm.at[pl.ds(sl * D_MODEL, D_MODEL)])

        if half:
            @pl.when(wid == nw - 1)
            def _():
                pltpu.sync_copy(tailp_hbm, obuf[1].at[pl.ds(0, half // 2)])
                pltpu.sync_copy(
                    obuf[1].at[pl.ds(0, half // 2)],
                    pr_hbm.at[pl.ds(full_slabs * D_MODEL, half // 2)],
                )

    @functools.partial(
        pl.kernel,
        mesh=plsc.VectorSubcoreMesh(**_MESH),
        compiler_params=pltpu.CompilerParams(
            use_tc_tiling_on_sc=True, needs_layout_passes=False
        ),
        out_type=jax.ShapeDtypeStruct((seq, D_MODEL, batch), jnp.float32),
        scratch_types=(
            [pltpu.VMEM((seq, CHUNK), jnp.int32)]
            + [pltpu.VMEM((NBUF, CHUNK), jnp.int32)]
            + [pltpu.VMEM((CHUNK, CHUNK), jnp.float32) for _ in range(NBUF)]
            + [pltpu.VMEM((D_MODEL, CHUNK), jnp.float32) for _ in range(NBUF)]
            + [pltpu.SemaphoreType.DMA for _ in range(2 * NBUF)]
        ),
    )
    def emb(x_hbm, tbl_hbm, out_hbm, idx_v, pidx_v, *bufs_and_sems):
        gbuf = bufs_and_sems[:NBUF]
        obuf = bufs_and_sems[NBUF : 2 * NBUF]
        gsem = bufs_and_sems[2 * NBUF : 3 * NBUF]
        ssem = bufs_and_sems[3 * NBUF : 4 * NBUF]
        wid = lax.axis_index("s") * nc + lax.axis_index("c")
        col0 = wid * CHUNK
        pltpu.sync_copy(x_hbm.at[:, pl.ds(col0, CHUNK)], idx_v)

        def start_gather(b, j):
            for g in range(CHUNK // LANES):
                s = pl.ds(g * LANES, LANES)
                pidx_v[b, s] = lax.shift_right_logical(idx_v[j, s], 1)
            pltpu.make_async_copy(
                tbl_hbm.at[pidx_v.at[b]], gbuf[b], gsem[b]
            ).start()

        def wait_gather(b):
            pltpu.make_async_copy(
                tbl_hbm.at[pidx_v.at[b]], gbuf[b], gsem[b]
            ).wait()

        def start_store(b, j):
            pltpu.make_async_copy(
                obuf[b], out_hbm.at[j, :, pl.ds(col0, CHUNK)], ssem[b]
            ).start()

        def wait_store(b):
            pltpu.make_async_copy(
                obuf[b], out_hbm.at[0, :, pl.ds(col0, CHUNK)], ssem[b]
            ).wait()

        for b in range(PF):
            start_gather(b, b)

        def outer(gi, carry):
            for b in range(NBUF):
                j = gi * NBUF + b
                jp = j + PF
                bp = (b + PF) % NBUF

                @pl.when(jp < seq)
                def _():
                    start_gather(bp, jp)

                wait_gather(b)

                @pl.when(j >= NBUF)
                def _():
                    wait_store(b)

                for g in range(CHUNK // LANES):
                    s = pl.ds(g * LANES, LANES)
                    row_ids = jax.lax.iota(jnp.int32, LANES) + (g * LANES)
                    col_base = (idx_v[j, s] & 1) * D_MODEL

                    @plsc.parallel_loop(0, D_MODEL, 1, unroll=4)
                    def _(k):
                        vals = plsc.load_gather(
                            gbuf[b], [row_ids, col_base + k]
                        )
                        obuf[b][k, s] = vals * SCALE

                start_store(b, j)
            return carry

        lax.fori_loop(0, seq // NBUF, outer, 0)
        for b in range(NBUF):
            wait_store(b)

    pair_rows = fmt(table_t, tail_pairs)
    return emb(x_t, pair_rows)


def kernel(x, table):
    b, s = x.shape
    vocab = table.shape[0]
    x_t = jnp.swapaxes(x, 0, 1).astype(jnp.int32)
    table_t = jnp.swapaxes(table, 0, 1)
    half = vocab - (vocab // CHUNK) * CHUNK
    tail_pairs = table[vocab - half :].reshape(half // 2, 2 * D_MODEL)
    out_t = _run(x_t, table_t, tail_pairs)
    return jnp.transpose(out_t, (2, 0, 1))


# speedup vs baseline: 2.0140x; 1.1884x over previous
import functools
import math

import jax
import jax.numpy as jnp
from jax import lax
from jax.experimental import pallas as pl
from jax.experimental.pallas import tpu as pltpu
from jax.experimental.pallas import tpu_sc as plsc

D_MODEL = 64
CHUNK = 128
LANES = 16
SCALE = math.sqrt(D_MODEL)
NBUF = 4
NOBUF = 2
PF = 2
SKEW = 1


def _splat(val):
    return jnp.full((LANES,), 0, jnp.int32) + val


@jax.jit
def _run(x_t, table_pad):
    info = plsc.get_sparse_core_info()
    nc, ns = info.num_cores, info.num_subcores
    seq, batch = x_t.shape

    mesh = plsc.VectorSubcoreMesh(core_axis_name="c", subcore_axis_name="s")

    @functools.partial(
        pl.kernel,
        mesh=mesh,
        compiler_params=pltpu.CompilerParams(
            use_tc_tiling_on_sc=True, needs_layout_passes=False
        ),
        out_type=jax.ShapeDtypeStruct((seq, D_MODEL, batch), jnp.float32),
        scratch_types=(
            [pltpu.VMEM((seq, CHUNK), jnp.int32)]
            + [pltpu.VMEM((CHUNK, D_MODEL + SKEW), jnp.float32)]
            + [pltpu.VMEM((CHUNK, CHUNK), jnp.float32) for _ in range(NBUF)]
            + [pltpu.VMEM((D_MODEL, CHUNK), jnp.float32) for _ in range(NOBUF)]
            + [pltpu.SemaphoreType.DMA for _ in range(NBUF + NOBUF)]
        ),
    )
    def emb(x_hbm, tbl_hbm, out_hbm, idx_v, sbuf, *bufs_and_sems):
        gbuf = bufs_and_sems[:NBUF]
        obuf = bufs_and_sems[NBUF : NBUF + NOBUF]
        gsem = bufs_and_sems[NBUF + NOBUF : 2 * NBUF + NOBUF]
        ssem = bufs_and_sems[2 * NBUF + NOBUF : 2 * NBUF + 2 * NOBUF]
        wid = lax.axis_index("s") * nc + lax.axis_index("c")
        col0 = wid * CHUNK
        pltpu.sync_copy(x_hbm.at[:, pl.ds(col0, CHUNK)], idx_v)

        def gather_copy(b, j):
            return pltpu.make_async_copy(
                tbl_hbm.at[idx_v.at[j]], gbuf[b], gsem[b]
            )

        def store_copy(b, j):
            return pltpu.make_async_copy(
                obuf[b], out_hbm.at[j, :, pl.ds(col0, CHUNK)], ssem[b]
            )

        for b in range(PF):
            gather_copy(b, b).start()

        def outer(gi, carry):
            for b in range(NBUF):
                j = gi * NBUF + b
                jp = j + PF
                bp = (b + PF) % NBUF

                @pl.when(jp < seq)
                def _():
                    gather_copy(bp, jp).start()

                gather_copy(b, j).wait()
                bo = b % NOBUF

                @pl.when(j >= NOBUF)
                def _():
                    store_copy(bo, j - NOBUF).wait()

                @plsc.parallel_loop(0, CHUNK, 1, unroll=4)
                def _(r):
                    for g in range(D_MODEL // LANES):
                        sl = pl.ds(g * LANES, LANES)
                        sbuf[r, sl] = gbuf[b][r, sl] * SCALE

                for g in range(CHUNK // LANES):
                    s = pl.ds(g * LANES, LANES)
                    row_ids = jax.lax.iota(jnp.int32, LANES) + (g * LANES)

                    @plsc.parallel_loop(0, D_MODEL, 1, unroll=4)
                    def _(k):
                        vals = plsc.load_gather(sbuf, [row_ids, _splat(k)])
                        obuf[bo][k, s] = vals

                store_copy(bo, j).start()
            return carry

        lax.fori_loop(0, seq // NBUF, outer, 0)
        for b in range(NOBUF):
            store_copy(b, seq - NOBUF + b).wait()

    return emb(x_t, table_pad)


def kernel(x, table):
    b, s = x.shape
    x_t = jnp.swapaxes(x, 0, 1).astype(jnp.int32)
    table_pad = jnp.pad(table, ((0, 0), (0, CHUNK - D_MODEL)))
    out_t = _run(x_t, table_pad)
    return jnp.transpose(out_t, (2, 0, 1))


# speedup vs baseline: 2.1355x; 1.0603x over previous
import functools
import math

import jax
import jax.numpy as jnp
from jax import lax
from jax.experimental import pallas as pl
from jax.experimental.pallas import tpu as pltpu
from jax.experimental.pallas import tpu_sc as plsc

D_MODEL = 64
CHUNK = 128
LANES = 16
SCALE = math.sqrt(D_MODEL)
NBUF = 4
NOBUF = 2
PF = 2
SKEW = 1


def _splat(val):
    return jnp.full((LANES,), 0, jnp.int32) + val


@jax.jit
def _run(x_t, table_pad):
    info = plsc.get_sparse_core_info()
    nc, ns = info.num_cores, info.num_subcores
    seq, batch = x_t.shape

    mesh = plsc.VectorSubcoreMesh(core_axis_name="c", subcore_axis_name="s")

    @functools.partial(
        pl.kernel,
        mesh=mesh,
        compiler_params=pltpu.CompilerParams(
            use_tc_tiling_on_sc=True, needs_layout_passes=False
        ),
        out_type=jax.ShapeDtypeStruct((seq, D_MODEL, batch), jnp.float32),
        scratch_types=(
            [pltpu.VMEM((seq, CHUNK), jnp.int32)]
            + [pltpu.VMEM((CHUNK, CHUNK), jnp.float32) for _ in range(NBUF)]
            + [pltpu.VMEM((D_MODEL, CHUNK), jnp.float32) for _ in range(NOBUF)]
            + [pltpu.SemaphoreType.DMA for _ in range(NBUF + NOBUF)]
        ),
    )
    def emb(x_hbm, tbl_hbm, out_hbm, idx_v, *bufs_and_sems):
        gbuf = bufs_and_sems[:NBUF]
        obuf = bufs_and_sems[NBUF : NBUF + NOBUF]
        gsem = bufs_and_sems[NBUF + NOBUF : 2 * NBUF + NOBUF]
        ssem = bufs_and_sems[2 * NBUF + NOBUF : 2 * NBUF + 2 * NOBUF]
        wid = lax.axis_index("s") * nc + lax.axis_index("c")
        col0 = wid * CHUNK
        pltpu.sync_copy(x_hbm.at[:, pl.ds(col0, CHUNK)], idx_v)

        def gather_copy(b, j):
            return pltpu.make_async_copy(
                tbl_hbm.at[idx_v.at[j]], gbuf[b], gsem[b]
            )

        def store_copy(b, j):
            return pltpu.make_async_copy(
                obuf[b], out_hbm.at[j, :, pl.ds(col0, CHUNK)], ssem[b]
            )

        for b in range(PF):
            gather_copy(b, b).start()

        def outer(gi, carry):
            for b in range(NBUF):
                j = gi * NBUF + b
                jp = j + PF
                bp = (b + PF) % NBUF

                @pl.when(jp < seq)
                def _():
                    gather_copy(bp, jp).start()

                gather_copy(b, j).wait()
                bo = b % NOBUF

                @pl.when(j >= NOBUF)
                def _():
                    store_copy(bo, j - NOBUF).wait()

                @plsc.parallel_loop(0, D_MODEL, 1, unroll=2)
                def _(k):
                    colv = _splat(k)
                    for g in range(CHUNK // LANES):
                        row_ids = jax.lax.iota(jnp.int32, LANES) + (g * LANES)
                        vals = plsc.load_gather(gbuf[b], [row_ids, colv])
                        obuf[bo][k, pl.ds(g * LANES, LANES)] = vals * SCALE

                store_copy(bo, j).start()
            return carry

        lax.fori_loop(0, seq // NBUF, outer, 0)
        for b in range(NOBUF):
            store_copy(b, seq - NOBUF + b).wait()

    return emb(x_t, table_pad)


def kernel(x, table):
    b, s = x.shape
    x_t = jnp.swapaxes(x, 0, 1).astype(jnp.int32)
    table_pad = jnp.pad(table, ((0, 0), (0, CHUNK - D_MODEL)))
    out_t = _run(x_t, table_pad)
    return jnp.transpose(out_t, (2, 0, 1))


# speedup vs baseline: 2.1387x; 1.0015x over previous
import functools
import math

import jax
import jax.numpy as jnp
from jax import lax
from jax.experimental import pallas as pl
from jax.experimental.pallas import tpu as pltpu
from jax.experimental.pallas import tpu_sc as plsc

D_MODEL = 64
CHUNK = 128
LANES = 16
SCALE = math.sqrt(D_MODEL)
NBUF = 4
NOBUF = 2
PF = 3
SKEW = 1


def _splat(val):
    return jnp.full((LANES,), 0, jnp.int32) + val


@jax.jit
def _run(x_t, table_pad):
    info = plsc.get_sparse_core_info()
    nc, ns = info.num_cores, info.num_subcores
    seq, batch = x_t.shape

    mesh = plsc.VectorSubcoreMesh(core_axis_name="c", subcore_axis_name="s")

    @functools.partial(
        pl.kernel,
        mesh=mesh,
        compiler_params=pltpu.CompilerParams(
            use_tc_tiling_on_sc=True, needs_layout_passes=False
        ),
        out_type=jax.ShapeDtypeStruct((seq, D_MODEL, batch), jnp.float32),
        scratch_types=(
            [pltpu.VMEM((seq, CHUNK), jnp.int32)]
            + [pltpu.VMEM((CHUNK, CHUNK), jnp.float32) for _ in range(NBUF)]
            + [pltpu.VMEM((D_MODEL, CHUNK), jnp.float32) for _ in range(NOBUF)]
            + [pltpu.SemaphoreType.DMA for _ in range(NBUF + NOBUF)]
        ),
    )
    def emb(x_hbm, tbl_hbm, out_hbm, idx_v, *bufs_and_sems):
        gbuf = bufs_and_sems[:NBUF]
        obuf = bufs_and_sems[NBUF : NBUF + NOBUF]
        gsem = bufs_and_sems[NBUF + NOBUF : 2 * NBUF + NOBUF]
        ssem = bufs_and_sems[2 * NBUF + NOBUF : 2 * NBUF + 2 * NOBUF]
        wid = lax.axis_index("s") * nc + lax.axis_index("c")
        col0 = wid * CHUNK
        pltpu.sync_copy(x_hbm.at[:, pl.ds(col0, CHUNK)], idx_v)

        def gather_copy(b, j):
            return pltpu.make_async_copy(
                tbl_hbm.at[idx_v.at[j]], gbuf[b], gsem[b]
            )

        def store_copy(b, j):
            return pltpu.make_async_copy(
                obuf[b], out_hbm.at[j, :, pl.ds(col0, CHUNK)], ssem[b]
            )

        for b in range(PF):
            gather_copy(b, b).start()

        def outer(gi, carry):
            for b in range(NBUF):
                j = gi * NBUF + b
                jp = j + PF
                bp = (b + PF) % NBUF

                @pl.when(jp < seq)
                def _():
                    gather_copy(bp, jp).start()

                gather_copy(b, j).wait()
                bo = b % NOBUF

                @pl.when(j >= NOBUF)
                def _():
                    store_copy(bo, j - NOBUF).wait()

                @plsc.parallel_loop(0, D_MODEL, 1, unroll=2)
                def _(k):
                    colv = _splat(k)
                    for g in range(CHUNK // LANES):
                        row_ids = jax.lax.iota(jnp.int32, LANES) + (g * LANES)
                        vals = plsc.load_gather(gbuf[b], [row_ids, colv])
                        obuf[bo][k, pl.ds(g * LANES, LANES)] = vals * SCALE

                store_copy(bo, j).start()
            return carry

        lax.fori_loop(0, seq // NBUF, outer, 0)
        for b in range(NOBUF):
            store_copy(b, seq - NOBUF + b).wait()

    return emb(x_t, table_pad)


def kernel(x, table):
    b, s = x.shape
    x_t = jnp.swapaxes(x, 0, 1).astype(jnp.int32)
    table_pad = lax.dynamic_update_slice(
        jnp.zeros((table.shape[0], CHUNK), jnp.float32), table, (0, 0)
    )
    out_t = _run(x_t, table_pad)
    return jnp.transpose(out_t, (2, 0, 1))
